# baseline (speedup 1.0000x reference)
import jax
import jax.numpy as jnp
import numpy as np
from jax.experimental import pallas as pl
from jax.experimental.pallas import tpu as pltpu

_B = 128
_N = 3 * 224 * 224
_NY = 1000

_key = jax.random.key(42)
_kb, _kp = jax.random.split(_key)
_COEFFS = np.asarray(jax.random.beta(_kb, 0.2, 0.2, (_B,)), np.float32)
_PERM = np.asarray(jax.random.permutation(_kp, _B), np.int32)


def _body(perm_ref, coef_ref, x_ref, xp_ref, y_ref, yp_ref, xo_ref, yo_ref):
    i = pl.program_id(0)
    c = coef_ref[i]
    xo_ref[...] = c * x_ref[...] + (1.0 - c) * xp_ref[...]
    yo_ref[...] = c * y_ref[...] + (1.0 - c) * yp_ref[...]


def kernel(X, Y):
    X3 = X.reshape(_B, 1, _N)
    Y3 = Y.reshape(_B, 1, _NY)
    grid_spec = pltpu.PrefetchScalarGridSpec(
        num_scalar_prefetch=2,
        grid=(_B,),
        in_specs=[
            pl.BlockSpec((1, 1, _N), lambda i, p, c: (i, 0, 0)),
            pl.BlockSpec((1, 1, _N), lambda i, p, c: (p[i], 0, 0)),
            pl.BlockSpec((1, 1, _NY), lambda i, p, c: (i, 0, 0)),
            pl.BlockSpec((1, 1, _NY), lambda i, p, c: (p[i], 0, 0)),
        ],
        out_specs=[
            pl.BlockSpec((1, 1, _N), lambda i, p, c: (i, 0, 0)),
            pl.BlockSpec((1, 1, _NY), lambda i, p, c: (i, 0, 0)),
        ],
    )
    Xo, Yo = pl.pallas_call(
        _body,
        grid_spec=grid_spec,
        out_shape=[
            jax.ShapeDtypeStruct((_B, 1, _N), X.dtype),
            jax.ShapeDtypeStruct((_B, 1, _NY), Y.dtype),
        ],
    )(jnp.asarray(_PERM), jnp.asarray(_COEFFS), X3, X3, Y3, Y3)
    return Xo.reshape(X.shape), Yo.reshape(Y.shape)


# speedup vs baseline: 1.0277x; 1.0277x over previous
import jax
import jax.numpy as jnp
import numpy as np
from jax.experimental import pallas as pl
from jax.experimental.pallas import tpu as pltpu

_B = 128
_N = 3 * 224 * 224
_NY = 1000

_key = jax.random.key(42)
_kb, _kp = jax.random.split(_key)
_COEFFS = np.asarray(jax.random.beta(_kb, 0.2, 0.2, (_B,)), np.float32)
_PERM = np.asarray(jax.random.permutation(_kp, _B), np.int32)


def _build_schedule(perm):
    seen = np.zeros(_B, bool)
    g, o = [], []
    for s in range(_B):
        if seen[s]:
            continue
        cyc = []
        j = s
        while not seen[j]:
            seen[j] = True
            cyc.append(j)
            j = int(perm[j])
        g.extend(cyc)
        g.append(cyc[0])
        o.append(cyc[0])
        o.extend(cyc)
    return (np.asarray(g, np.int32), np.asarray(o, np.int32))


_G, _O = _build_schedule(_PERM)
_S = len(_G)
_CF = _COEFFS[_O]
_PMAT = np.zeros((_B, _B), np.float32)
_PMAT[np.arange(_B), _PERM] = 1.0


def _body(g_ref, o_ref, cf_ref, xg_ref, y_ref, p_ref, cy_ref,
          xo_ref, yo_ref, xs_ref):
    t = pl.program_id(0)
    c = cf_ref[t]
    xo_ref[...] = c * xs_ref[...] + (1.0 - c) * xg_ref[...]
    xs_ref[...] = xg_ref[...]

    @pl.when(t == 0)
    def _():
        y = y_ref[...]
        yp = jnp.dot(p_ref[...], y, preferred_element_type=jnp.float32)
        cy = cy_ref[...]
        yo_ref[...] = cy * y + (1.0 - cy) * yp


def kernel(X, Y):
    X3 = X.reshape(_B, 1, _N)
    grid_spec = pltpu.PrefetchScalarGridSpec(
        num_scalar_prefetch=3,
        grid=(_S,),
        in_specs=[
            pl.BlockSpec((1, 1, _N), lambda t, g, o, cf: (g[t], 0, 0)),
            pl.BlockSpec((_B, _NY), lambda t, g, o, cf: (0, 0)),
            pl.BlockSpec((_B, _B), lambda t, g, o, cf: (0, 0)),
            pl.BlockSpec((_B, 1), lambda t, g, o, cf: (0, 0)),
        ],
        out_specs=[
            pl.BlockSpec((1, 1, _N), lambda t, g, o, cf: (o[t], 0, 0)),
            pl.BlockSpec((_B, _NY), lambda t, g, o, cf: (0, 0)),
        ],
        scratch_shapes=[pltpu.VMEM((1, 1, _N), jnp.float32)],
    )
    Xo, Yo = pl.pallas_call(
        _body,
        grid_spec=grid_spec,
        out_shape=[
            jax.ShapeDtypeStruct((_B, 1, _N), X.dtype),
            jax.ShapeDtypeStruct((_B, _NY), Y.dtype),
        ],
    )(jnp.asarray(_G), jnp.asarray(_O), jnp.asarray(_CF),
      X3, Y, jnp.asarray(_PMAT), jnp.asarray(_COEFFS.reshape(_B, 1)))
    return Xo.reshape(X.shape), Yo


# speedup vs baseline: 1.8409x; 1.7913x over previous
import jax
import jax.numpy as jnp
import numpy as np
from jax.experimental import pallas as pl
from jax.experimental.pallas import tpu as pltpu

_B = 128
_N = 3 * 224 * 224
_NY = 1000

_key = jax.random.key(42)
_kb, _kp = jax.random.split(_key)
_COEFFS = np.asarray(jax.random.beta(_kb, 0.2, 0.2, (_B,)), np.float32)
_PERM = np.asarray(jax.random.permutation(_kp, _B), np.int32)


def _build_schedule(perm):
    seen = np.zeros(_B, bool)
    g, o = [], []
    for s in range(_B):
        if seen[s]:
            continue
        cyc = []
        j = s
        while not seen[j]:
            seen[j] = True
            cyc.append(j)
            j = int(perm[j])
        g.extend(cyc)
        g.append(cyc[0])
        o.append(cyc[0])
        o.extend(cyc)
    return (np.asarray(g, np.int32), np.asarray(o, np.int32))


_G, _O = _build_schedule(_PERM)
_S = len(_G)
_CF = _COEFFS[_O]
_PMAT = np.zeros((_B, _B), np.float32)
_PMAT[np.arange(_B), _PERM] = 1.0


def _body(g_ref, o_ref, cf_ref, xg_ref, y_ref, p_ref, cy_ref,
          xo_ref, yo_ref, xs_ref):
    t = pl.program_id(0)
    c = cf_ref[t]
    xo_ref[...] = c * xs_ref[...] + (1.0 - c) * xg_ref[...]
    xs_ref[...] = xg_ref[...]

    @pl.when(t == 0)
    def _():
        y = y_ref[...]
        yp = jnp.dot(p_ref[...], y, preferred_element_type=jnp.float32)
        cy = cy_ref[...]
        yo_ref[...] = cy * y + (1.0 - cy) * yp


def kernel(X, Y):
    X3 = X.reshape(_B, _N // 128, 128)
    grid_spec = pltpu.PrefetchScalarGridSpec(
        num_scalar_prefetch=3,
        grid=(_S,),
        in_specs=[
            pl.BlockSpec((1, _N // 128, 128), lambda t, g, o, cf: (g[t], 0, 0)),
            pl.BlockSpec((_B, _NY), lambda t, g, o, cf: (0, 0)),
            pl.BlockSpec((_B, _B), lambda t, g, o, cf: (0, 0)),
            pl.BlockSpec((_B, 1), lambda t, g, o, cf: (0, 0)),
        ],
        out_specs=[
            pl.BlockSpec((1, _N // 128, 128), lambda t, g, o, cf: (o[t], 0, 0)),
            pl.BlockSpec((_B, _NY), lambda t, g, o, cf: (0, 0)),
        ],
        scratch_shapes=[pltpu.VMEM((1, _N // 128, 128), jnp.float32)],
    )
    Xo, Yo = pl.pallas_call(
        _body,
        grid_spec=grid_spec,
        out_shape=[
            jax.ShapeDtypeStruct((_B, _N // 128, 128), X.dtype),
            jax.ShapeDtypeStruct((_B, _NY), Y.dtype),
        ],
    )(jnp.asarray(_G), jnp.asarray(_O), jnp.asarray(_CF),
      X3, Y, jnp.asarray(_PMAT), jnp.asarray(_COEFFS.reshape(_B, 1)))
    return Xo.reshape(X.shape), Yo


# speedup vs baseline: 1.8472x; 1.0034x over previous
import jax
import jax.numpy as jnp
import numpy as np
from jax.experimental import pallas as pl
from jax.experimental.pallas import tpu as pltpu

_B = 128
_N = 3 * 224 * 224
_R = _N // 128
_NY = 1000

_key = jax.random.key(42)
_kb, _kp = jax.random.split(_key)
_COEFFS = np.asarray(jax.random.beta(_kb, 0.2, 0.2, (_B,)), np.float32)
_PERM = np.asarray(jax.random.permutation(_kp, _B), np.int32)


def _build_schedule(perm):
    seen = np.zeros(_B, bool)
    g, o = [], []
    for s in range(_B):
        if seen[s]:
            continue
        cyc = []
        j = s
        while not seen[j]:
            seen[j] = True
            cyc.append(j)
            j = int(perm[j])
        g.extend(cyc)
        g.append(cyc[0])
        o.append(cyc[0])
        o.extend(cyc)
    return (np.asarray(g, np.int32), np.asarray(o, np.int32))


_G, _O = _build_schedule(_PERM)
_S = len(_G)
_CF = _COEFFS[_O]
_PMAT = np.zeros((_B, _B), np.float32)
_PMAT[np.arange(_B), _PERM] = 1.0


def _body(g_ref, o_ref, cf_ref, xa_ref, xb_ref, y_ref, p_ref, cy_ref,
          xo_ref, yo_ref):
    t = pl.program_id(0)
    c = cf_ref[t]

    @pl.when(t % 2 == 0)
    def _():
        xo_ref[...] = c * xb_ref[...] + (1.0 - c) * xa_ref[...]

    @pl.when(t % 2 == 1)
    def _():
        xo_ref[...] = c * xa_ref[...] + (1.0 - c) * xb_ref[...]

    @pl.when(t == 0)
    def _():
        y = y_ref[...]
        yp = jnp.dot(p_ref[...], y, preferred_element_type=jnp.float32)
        cy = cy_ref[...]
        yo_ref[...] = cy * y + (1.0 - cy) * yp


def _idx_a(t, g, o, cf):
    return (g[t - (t % 2)], 0, 0)


def _idx_b(t, g, o, cf):
    u = jnp.maximum(t - ((t + 1) % 2), 0)
    return (g[u], 0, 0)


def kernel(X, Y):
    X3 = X.reshape(_B, _R, 128)
    grid_spec = pltpu.PrefetchScalarGridSpec(
        num_scalar_prefetch=3,
        grid=(_S,),
        in_specs=[
            pl.BlockSpec((1, _R, 128), _idx_a),
            pl.BlockSpec((1, _R, 128), _idx_b),
            pl.BlockSpec((_B, _NY), lambda t, g, o, cf: (0, 0)),
            pl.BlockSpec((_B, _B), lambda t, g, o, cf: (0, 0)),
            pl.BlockSpec((_B, 1), lambda t, g, o, cf: (0, 0)),
        ],
        out_specs=[
            pl.BlockSpec((1, _R, 128), lambda t, g, o, cf: (o[t], 0, 0)),
            pl.BlockSpec((_B, _NY), lambda t, g, o, cf: (0, 0)),
        ],
    )
    Xo, Yo = pl.pallas_call(
        _body,
        grid_spec=grid_spec,
        out_shape=[
            jax.ShapeDtypeStruct((_B, _R, 128), X.dtype),
            jax.ShapeDtypeStruct((_B, _NY), Y.dtype),
        ],
    )(jnp.asarray(_G), jnp.asarray(_O), jnp.asarray(_CF),
      X3, X3, Y, jnp.asarray(_PMAT), jnp.asarray(_COEFFS.reshape(_B, 1)))
    return Xo.reshape(X.shape), Yo
